# Initial kernel scaffold; baseline (speedup 1.0000x reference)
import jax
import jax.numpy as jnp
from jax.experimental import pallas as pl


def kernel(z, pos, batch, edge_index, emb_table, mlp_w1, mlp_b1, mlp_w2, mlp_b2, conv1_w, conv2_w, conv2_b, lin_w, lin_b, pool_w, pool_b):
    raise NotImplementedError("write your pallas kernel here")


# speedup vs baseline: 1.7824x; 1.7824x over previous
import functools
import math

import jax
import jax.numpy as jnp
from jax import lax
from jax.experimental import pallas as pl
from jax.experimental.pallas import tpu as pltpu
from jax.experimental.pallas import tpu_sc as plsc

HIDDEN = 256
NUM_INTER = 6
NUM_GAUSS = 50
CUTOFF = 10.0
N_NODES = 10000
N_EDGES = 160000
NUM_GRAPHS = 64

NC = 2
NS = 16
LANES = 16
HALF = HIDDEN // 2
N_EP = 163840
CHUNK = 128
AGG_ROWS = 10240
_LN2 = math.log(2.0)


def _ssp(x):
    return jnp.maximum(x, 0.0) + jnp.log(1.0 + jnp.exp(-jnp.abs(x))) - _LN2


def _sc_geom(pos_p, src_p, dst_p):
    ept = N_EP // (NC * NS)
    mesh = plsc.VectorSubcoreMesh(core_axis_name="c", subcore_axis_name="s")

    @functools.partial(
        pl.kernel,
        out_type=jax.ShapeDtypeStruct((N_EP,), jnp.float32),
        mesh=mesh,
        compiler_params=pltpu.CompilerParams(needs_layout_passes=False),
    scratch_types=[
            pltpu.VMEM((4 * N_NODES,), jnp.float32),
            pltpu.VMEM((ept,), jnp.int32),
            pltpu.VMEM((ept,), jnp.int32),
            pltpu.VMEM((ept,), jnp.float32),
        ],
    )
    def k(pos_hbm, src_hbm, dst_hbm, out_hbm, pos_v, src_v, dst_v, out_v):
        cid = lax.axis_index("c")
        sid = lax.axis_index("s")
        wid = sid * NC + cid
        base = wid * ept
        pltpu.sync_copy(pos_hbm, pos_v)
        pltpu.sync_copy(src_hbm.at[pl.ds(base, ept)], src_v)
        pltpu.sync_copy(dst_hbm.at[pl.ds(base, ept)], dst_v)

        def body(g, carry):
            sl = pl.ds(g * LANES, LANES)
            sv = src_v[sl] * 4
            dv = dst_v[sl] * 4
            acc = jnp.full((LANES,), 1e-12, jnp.float32)
            for c in range(3):
                a = plsc.load_gather(pos_v, [sv + c])
                b = plsc.load_gather(pos_v, [dv + c])
                d = a - b
                acc = acc + d * d
            out_v[sl] = acc
            return carry

        lax.fori_loop(0, ept // LANES, body, 0)
        pltpu.sync_copy(out_v, out_hbm.at[pl.ds(base, ept)])

    return k(pos_p, src_p, dst_p)


def _sc_cfconv(xf_cat, w_cat, src2, dst_p):
    ept = N_EP // NS
    nch = ept // CHUNK
    zstripe = AGG_ROWS // NS
    ostripe = (N_NODES // NS) // 8 * 8
    mesh = plsc.VectorSubcoreMesh(core_axis_name="c", subcore_axis_name="s")

    @functools.partial(
        pl.kernel,
        out_type=jax.ShapeDtypeStruct((2 * N_NODES, HALF), jnp.float32),
        mesh=mesh,
        compiler_params=pltpu.CompilerParams(needs_layout_passes=False),
        scratch_types=[
            pltpu.VMEM((CHUNK,), jnp.int32),
            pltpu.VMEM((CHUNK,), jnp.int32),
            pltpu.VMEM((CHUNK, HALF), jnp.float32),
            pltpu.VMEM((CHUNK, HALF), jnp.float32),
            pltpu.VMEM_SHARED((AGG_ROWS, HALF), jnp.float32),
            pltpu.SemaphoreType.DMA,
        ],
    )
    def k(xf_hbm, w_hbm, src_hbm, dst_hbm, out_hbm,
          src_v, dst_v, rows_v, w_v, agg_sh, sem):
        cid = lax.axis_index("c")
        sid = lax.axis_index("s")
        ebase = sid * ept
        gbase = cid * N_EP + ebase

        zvec = jnp.zeros((LANES,), jnp.float32)

        def zbody(j, carry):
            def zrow(f, carry2):
                rows_v[j, pl.ds(f * LANES, LANES)] = zvec
                return carry2
            lax.fori_loop(0, HALF // LANES, zrow, 0)
            return carry

        lax.fori_loop(0, CHUNK, zbody, 0)
        for q in range(zstripe // CHUNK):
            pltpu.sync_copy(
                rows_v, agg_sh.at[pl.ds(sid * zstripe + q * CHUNK, CHUNK)])
        plsc.subcore_barrier()

        def chunk_body(kk, carry):
            off = kk * CHUNK
            pltpu.sync_copy(src_hbm.at[pl.ds(gbase + off, CHUNK)], src_v)
            pltpu.sync_copy(dst_hbm.at[pl.ds(ebase + off, CHUNK)], dst_v)
            pltpu.async_copy(xf_hbm.at[src_v], rows_v, sem).wait()
            pltpu.sync_copy(w_hbm.at[pl.ds(gbase + off, CHUNK)], w_v)

            def mul(j, carry2):
                for f in range(HALF // LANES):
                    sl = pl.ds(f * LANES, LANES)
                    rows_v[j, sl] = rows_v[j, sl] * w_v[j, sl]
                return carry2

            lax.fori_loop(0, CHUNK, mul, 0)
            pltpu.sync_copy(rows_v, agg_sh.at[dst_v], add=True)
            return carry

        lax.fori_loop(0, nch, chunk_body, 0)
        plsc.subcore_barrier()

        pltpu.sync_copy(
            agg_sh.at[pl.ds(sid * ostripe, ostripe)],
            out_hbm.at[pl.ds(cid * N_NODES + sid * ostripe, ostripe)])

        @pl.when(sid == 0)
        def _tail():
            pltpu.sync_copy(
                agg_sh.at[pl.ds(NS * ostripe, N_NODES - NS * ostripe)],
                out_hbm.at[pl.ds(cid * N_NODES + NS * ostripe,
                                 N_NODES - NS * ostripe)])

    return k(xf_cat, w_cat, src2, dst_p)


_EB = 2048
_NB = 2000


def _tc_edge_feats(d2_2d):
    step = CUTOFF / (NUM_GAUSS - 1)
    coeff = -0.5 / step ** 2

    def body(d2_ref, out_ref):
        d2 = d2_ref[...]
        dist = jnp.sqrt(d2)
        kii = lax.broadcasted_iota(jnp.int32, (_EB, 64), 1)
        kidx = kii.astype(jnp.float32)
        gauss = jnp.exp(coeff * (dist - kidx * step) ** 2)
        cc = 0.5 * (jnp.cos(dist * (math.pi / CUTOFF)) + 1.0)
        out_ref[...] = jnp.where(kii == NUM_GAUSS, cc, gauss)

    return pl.pallas_call(
        body,
        grid=(N_EP // _EB,),
        in_specs=[pl.BlockSpec((_EB, 1), lambda i: (i, 0))],
        out_specs=pl.BlockSpec((_EB, 64), lambda i: (i, 0)),
        out_shape=jax.ShapeDtypeStruct((N_EP, 64), jnp.float32),
    )(d2_2d)


def _tc_edge_mlp(eac, w1p, b1, w2, b2):
    def body(ea_ref, w1_ref, b1_ref, w2_ref, b2_ref, out_ref):
        ea = ea_ref[...]
        h1 = jnp.dot(ea, w1_ref[...], preferred_element_type=jnp.float32)
        h1 = _ssp(h1 + b1_ref[...])
        w = jnp.dot(h1, w2_ref[...], preferred_element_type=jnp.float32)
        w = (w + b2_ref[...]) * ea[:, NUM_GAUSS:NUM_GAUSS + 1]
        out_ref[0] = w[:, :HALF]
        out_ref[1] = w[:, HALF:]

    return pl.pallas_call(
        body,
        grid=(N_EP // _EB,),
        in_specs=[
            pl.BlockSpec((_EB, 64), lambda i: (i, 0)),
            pl.BlockSpec((64, HIDDEN), lambda i: (0, 0)),
            pl.BlockSpec((1, HIDDEN), lambda i: (0, 0)),
            pl.BlockSpec((HIDDEN, HIDDEN), lambda i: (0, 0)),
            pl.BlockSpec((1, HIDDEN), lambda i: (0, 0)),
        ],
        out_specs=pl.BlockSpec((2, _EB, HALF), lambda i: (0, i, 0)),
        out_shape=jax.ShapeDtypeStruct((2, N_EP, HALF), jnp.float32),
    )(eac, w1p, b1, w2, b2)


def _tc_emb(z_2d, emb_pad):
    def body(z_ref, emb_ref, out_ref):
        bi = lax.broadcasted_iota(jnp.int32, (_NB, 128), 1)
        oh = (bi == z_ref[...]).astype(jnp.float32)
        out_ref[...] = jnp.dot(oh, emb_ref[...],
                               preferred_element_type=jnp.float32)

    return pl.pallas_call(
        body,
        grid=(N_NODES // _NB,),
        in_specs=[
            pl.BlockSpec((_NB, 1), lambda i: (i, 0)),
            pl.BlockSpec((128, HIDDEN), lambda i: (0, 0)),
        ],
        out_specs=pl.BlockSpec((_NB, HIDDEN), lambda i: (i, 0)),
        out_shape=jax.ShapeDtypeStruct((N_NODES, HIDDEN), jnp.float32),
    )(z_2d, emb_pad)


def _tc_xf(h, w):
    def body(h_ref, w_ref, out_ref):
        xf = jnp.dot(h_ref[...], w_ref[...],
                     preferred_element_type=jnp.float32)
        out_ref[0] = xf[:, :HALF]
        out_ref[1] = xf[:, HALF:]

    return pl.pallas_call(
        body,
        grid=(N_NODES // _NB,),
        in_specs=[
            pl.BlockSpec((_NB, HIDDEN), lambda i: (i, 0)),
            pl.BlockSpec((HIDDEN, HIDDEN), lambda i: (0, 0)),
        ],
        out_specs=pl.BlockSpec((2, _NB, HALF), lambda i: (0, i, 0)),
        out_shape=jax.ShapeDtypeStruct((2, N_NODES, HALF), jnp.float32),
    )(h, w)


def _tc_out_mlp(agg3, h, w2, b2, lw, lb):
    def body(agg_ref, h_ref, w2_ref, b2_ref, lw_ref, lb_ref, out_ref):
        t = jnp.dot(agg_ref[0], w2_ref[:HALF, :],
                    preferred_element_type=jnp.float32)
        t = t + jnp.dot(agg_ref[1], w2_ref[HALF:, :],
                        preferred_element_type=jnp.float32)
        t = _ssp(t + b2_ref[...])
        t = jnp.dot(t, lw_ref[...], preferred_element_type=jnp.float32)
        out_ref[...] = h_ref[...] + t + lb_ref[...]

    return pl.pallas_call(
        body,
        grid=(N_NODES // _NB,),
        in_specs=[
            pl.BlockSpec((2, _NB, HALF), lambda i: (0, i, 0)),
            pl.BlockSpec((_NB, HIDDEN), lambda i: (i, 0)),
            pl.BlockSpec((HIDDEN, HIDDEN), lambda i: (0, 0)),
            pl.BlockSpec((1, HIDDEN), lambda i: (0, 0)),
            pl.BlockSpec((HIDDEN, HIDDEN), lambda i: (0, 0)),
            pl.BlockSpec((1, HIDDEN), lambda i: (0, 0)),
        ],
        out_specs=pl.BlockSpec((_NB, HIDDEN), lambda i: (i, 0)),
        out_shape=jax.ShapeDtypeStruct((N_NODES, HIDDEN), jnp.float32),
    )(agg3, h, w2, b2, lw, lb)


def _tc_pool(h, batch_row, pool_w, pool_b):
    nsteps = N_NODES // _NB

    def body(batch_ref, h_ref, pw_ref, pb_ref, out_ref, sums, cnt):
        i = pl.program_id(0)

        @pl.when(i == 0)
        def _init():
            sums[...] = jnp.zeros((NUM_GRAPHS, HIDDEN), jnp.float32)
            cnt[...] = jnp.zeros((NUM_GRAPHS, 128), jnp.float32)

        gi = lax.broadcasted_iota(jnp.int32, (NUM_GRAPHS, _NB), 0)
        oh = (gi == batch_ref[0]).astype(jnp.float32)
        sums[...] += jnp.dot(oh, h_ref[...],
                             preferred_element_type=jnp.float32)
        cnt[...] += jnp.dot(oh, jnp.ones((_NB, 128), jnp.float32),
                            preferred_element_type=jnp.float32)

        @pl.when(i == nsteps - 1)
        def _fin():
            pooled = sums[...] / jnp.maximum(cnt[:, 0:1], 1.0)
            out_ref[...] = jnp.dot(pooled, pw_ref[...],
                                   preferred_element_type=jnp.float32) \
                + pb_ref[...]

    return pl.pallas_call(
        body,
        grid=(nsteps,),
        in_specs=[
            pl.BlockSpec((1, 1, _NB), lambda i: (i, 0, 0)),
            pl.BlockSpec((_NB, HIDDEN), lambda i: (i, 0)),
            pl.BlockSpec((HIDDEN, HIDDEN), lambda i: (0, 0)),
            pl.BlockSpec((1, HIDDEN), lambda i: (0, 0)),
        ],
        out_specs=pl.BlockSpec((NUM_GRAPHS, HIDDEN), lambda i: (0, 0)),
        out_shape=jax.ShapeDtypeStruct((NUM_GRAPHS, HIDDEN), jnp.float32),
        scratch_shapes=[
            pltpu.VMEM((NUM_GRAPHS, HIDDEN), jnp.float32),
            pltpu.VMEM((NUM_GRAPHS, 128), jnp.float32),
        ],
    )(batch_row, h, pool_w, pool_b)


def kernel(z, pos, batch, edge_index, emb_table, mlp_w1, mlp_b1, mlp_w2,
           mlp_b2, conv1_w, conv2_w, conv2_b, lin_w, lin_b, pool_w, pool_b):
    src = edge_index[0].astype(jnp.int32)
    dst = edge_index[1].astype(jnp.int32)
    npad = N_EP - N_EDGES
    src_p = jnp.concatenate([src, jnp.zeros((npad,), jnp.int32)])
    dst_p = jnp.concatenate([dst, jnp.full((npad,), N_NODES, jnp.int32)])
    src2 = jnp.concatenate([src_p, src_p + N_NODES])
    pos_p = jnp.pad(pos.astype(jnp.float32), ((0, 0), (0, 1))).reshape(-1)

    d2 = _sc_geom(pos_p, src_p, dst_p)
    eac = _tc_edge_feats(d2.reshape(N_EP, 1))

    emb_pad = jnp.pad(emb_table, ((0, 128 - emb_table.shape[0]), (0, 0)))
    h = _tc_emb(z.reshape(N_NODES, 1).astype(jnp.int32), emb_pad)

    w1p = jnp.pad(mlp_w1, ((0, 0), (0, 64 - NUM_GAUSS), (0, 0)))

    for i in range(NUM_INTER):
        w3 = _tc_edge_mlp(eac, w1p[i], mlp_b1[i].reshape(1, HIDDEN),
                          mlp_w2[i], mlp_b2[i].reshape(1, HIDDEN))
        xf3 = _tc_xf(h, conv1_w[i])
        agg = _sc_cfconv(xf3.reshape(2 * N_NODES, HALF),
                         w3.reshape(2 * N_EP, HALF), src2, dst_p)
        h = _tc_out_mlp(agg.reshape(2, N_NODES, HALF), h, conv2_w[i],
                        conv2_b[i].reshape(1, HIDDEN), lin_w[i],
                        lin_b[i].reshape(1, HIDDEN))

    return _tc_pool(h, batch.reshape(N_NODES // _NB, 1, _NB).astype(jnp.int32),
                    pool_w, pool_b.reshape(1, HIDDEN))


# speedup vs baseline: 2.3710x; 1.3302x over previous
import functools
import math

import jax
import jax.numpy as jnp
from jax import lax
from jax.experimental import pallas as pl
from jax.experimental.pallas import tpu as pltpu
from jax.experimental.pallas import tpu_sc as plsc

HIDDEN = 256
NUM_INTER = 6
NUM_GAUSS = 50
CUTOFF = 10.0
N_NODES = 10000
N_EDGES = 160000
NUM_GRAPHS = 64

NC = 2
NS = 16
LANES = 16
HALF = HIDDEN // 2
N_EP = 163840
CHUNK = 80
AGG_ROWS = 10240
_LN2 = math.log(2.0)


def _ssp(x):
    return jnp.maximum(x, 0.0) + jnp.log(1.0 + jnp.exp(-jnp.abs(x))) - _LN2


def _sc_geom(pos_p, src_p, dst_p):
    ept = N_EP // (NC * NS)
    mesh = plsc.VectorSubcoreMesh(core_axis_name="c", subcore_axis_name="s")

    @functools.partial(
        pl.kernel,
        out_type=jax.ShapeDtypeStruct((N_EP,), jnp.float32),
        mesh=mesh,
        compiler_params=pltpu.CompilerParams(needs_layout_passes=False),
    scratch_types=[
            pltpu.VMEM((4 * N_NODES,), jnp.float32),
            pltpu.VMEM((ept,), jnp.int32),
            pltpu.VMEM((ept,), jnp.int32),
            pltpu.VMEM((ept,), jnp.float32),
        ],
    )
    def k(pos_hbm, src_hbm, dst_hbm, out_hbm, pos_v, src_v, dst_v, out_v):
        cid = lax.axis_index("c")
        sid = lax.axis_index("s")
        wid = sid * NC + cid
        base = wid * ept
        pltpu.sync_copy(pos_hbm, pos_v)
        pltpu.sync_copy(src_hbm.at[pl.ds(base, ept)], src_v)
        pltpu.sync_copy(dst_hbm.at[pl.ds(base, ept)], dst_v)

        def body(g, carry):
            sl = pl.ds(g * LANES, LANES)
            sv = src_v[sl] * 4
            dv = dst_v[sl] * 4
            acc = jnp.full((LANES,), 1e-12, jnp.float32)
            for c in range(3):
                a = plsc.load_gather(pos_v, [sv + c])
                b = plsc.load_gather(pos_v, [dv + c])
                d = a - b
                acc = acc + d * d
            out_v[sl] = acc
            return carry

        lax.fori_loop(0, ept // LANES, body, 0)
        pltpu.sync_copy(out_v, out_hbm.at[pl.ds(base, ept)])

    return k(pos_p, src_p, dst_p)


def _sc_cfconv(xf_cat, w_cat, src2_2d, dst_2d):
    ept = N_EP // NS
    nch = ept // CHUNK
    zstripe = AGG_ROWS // NS
    ostripe = (N_NODES // NS) // 8 * 8
    mesh = plsc.VectorSubcoreMesh(core_axis_name="c", subcore_axis_name="s")

    @functools.partial(
        pl.kernel,
        out_type=jax.ShapeDtypeStruct((2 * N_NODES, HALF), jnp.float32),
        mesh=mesh,
        compiler_params=pltpu.CompilerParams(needs_layout_passes=False),
        scratch_types=[
            pltpu.VMEM((2, CHUNK), jnp.int32),
            pltpu.VMEM((2, CHUNK), jnp.int32),
            pltpu.VMEM((CHUNK, HALF), jnp.float32),
            pltpu.VMEM((CHUNK, HALF), jnp.float32),
            pltpu.VMEM((CHUNK, HALF), jnp.float32),
            pltpu.VMEM((CHUNK, HALF), jnp.float32),
            pltpu.VMEM_SHARED((AGG_ROWS, HALF), jnp.float32),
            pltpu.SemaphoreType.DMA,
            pltpu.SemaphoreType.DMA,
            pltpu.SemaphoreType.DMA,
            pltpu.SemaphoreType.DMA,
        ],
    )
    def k(xf_hbm, w_hbm, src_hbm, dst_hbm, out_hbm,
          srcb, dstb, rows_a, rows_b, w_a, w_b, agg_sh,
          sga, sgb, swa, swb):
        cid = lax.axis_index("c")
        sid = lax.axis_index("s")
        irow = sid * nch
        grow = cid * (N_EP // CHUNK) + irow
        gbase = cid * N_EP + sid * ept

        zvec = jnp.zeros((LANES,), jnp.float32)

        def zbody(j, carry):
            for f in range(HALF // LANES):
                rows_a[j, pl.ds(f * LANES, LANES)] = zvec
            return carry

        lax.fori_loop(0, CHUNK, zbody, 0)
        for q in range(zstripe // CHUNK):
            pltpu.sync_copy(
                rows_a, agg_sh.at[pl.ds(sid * zstripe + q * CHUNK, CHUNK)])
        plsc.subcore_barrier()

        bufs = ((rows_a, w_a, sga, swa), (rows_b, w_b, sgb, swb))

        def load_idx_and_issue(kk, b):
            rv, wv, sg, sw = bufs[b]
            pltpu.sync_copy(src_hbm.at[grow + kk], srcb.at[b])
            pltpu.sync_copy(dst_hbm.at[irow + kk], dstb.at[b])
            pltpu.async_copy(xf_hbm.at[srcb.at[b]], rv, sg)
            pltpu.async_copy(w_hbm.at[pl.ds(gbase + kk * CHUNK, CHUNK)],
                             wv, sw)

        def consume(kk, b):
            rv, wv, sg, sw = bufs[b]
            pltpu.make_async_copy(xf_hbm.at[srcb.at[b]], rv, sg).wait()
            pltpu.make_async_copy(
                w_hbm.at[pl.ds(gbase + kk * CHUNK, CHUNK)], wv, sw).wait()

            def mul(j, carry2):
                for f in range(HALF // LANES):
                    sl = pl.ds(f * LANES, LANES)
                    rv[j, sl] = rv[j, sl] * wv[j, sl]
                return carry2

            lax.fori_loop(0, CHUNK, mul, 0)
            pltpu.sync_copy(rv, agg_sh.at[dstb.at[b]], add=True)

        load_idx_and_issue(0, 0)
        load_idx_and_issue(1, 1)

        def pair_body(i, carry):
            c0 = 2 * i
            consume(c0, 0)

            @pl.when(c0 + 2 < nch)
            def _n0():
                load_idx_and_issue(c0 + 2, 0)

            consume(c0 + 1, 1)

            @pl.when(c0 + 3 < nch)
            def _n1():
                load_idx_and_issue(c0 + 3, 1)

            return carry

        lax.fori_loop(0, nch // 2, pair_body, 0)
        plsc.subcore_barrier()

        pltpu.sync_copy(
            agg_sh.at[pl.ds(sid * ostripe, ostripe)],
            out_hbm.at[pl.ds(cid * N_NODES + sid * ostripe, ostripe)])

        @pl.when(sid == 0)
        def _tail():
            pltpu.sync_copy(
                agg_sh.at[pl.ds(NS * ostripe, N_NODES - NS * ostripe)],
                out_hbm.at[pl.ds(cid * N_NODES + NS * ostripe,
                                 N_NODES - NS * ostripe)])

    return k(xf_cat, w_cat, src2_2d, dst_2d)


_EB = 2048
_NB = 2000


def _tc_edge_feats(d2_2d):
    step = CUTOFF / (NUM_GAUSS - 1)
    coeff = -0.5 / step ** 2

    def body(d2_ref, out_ref):
        d2 = d2_ref[...]
        dist = jnp.sqrt(d2)
        kii = lax.broadcasted_iota(jnp.int32, (_EB, 64), 1)
        kidx = kii.astype(jnp.float32)
        gauss = jnp.exp(coeff * (dist - kidx * step) ** 2)
        cc = 0.5 * (jnp.cos(dist * (math.pi / CUTOFF)) + 1.0)
        out_ref[...] = jnp.where(kii == NUM_GAUSS, cc, gauss)

    return pl.pallas_call(
        body,
        grid=(N_EP // _EB,),
        in_specs=[pl.BlockSpec((_EB, 1), lambda i: (i, 0))],
        out_specs=pl.BlockSpec((_EB, 64), lambda i: (i, 0)),
        out_shape=jax.ShapeDtypeStruct((N_EP, 64), jnp.float32),
    )(d2_2d)


def _tc_edge_mlp(eac, w1p, b1, w2, b2):
    def body(ea_ref, w1_ref, b1_ref, w2_ref, b2_ref, out_ref):
        ea = ea_ref[...]
        h1 = jnp.dot(ea, w1_ref[...], preferred_element_type=jnp.float32)
        h1 = _ssp(h1 + b1_ref[...])
        w = jnp.dot(h1, w2_ref[...], preferred_element_type=jnp.float32)
        w = (w + b2_ref[...]) * ea[:, NUM_GAUSS:NUM_GAUSS + 1]
        out_ref[0] = w[:, :HALF]
        out_ref[1] = w[:, HALF:]

    return pl.pallas_call(
        body,
        grid=(N_EP // _EB,),
        in_specs=[
            pl.BlockSpec((_EB, 64), lambda i: (i, 0)),
            pl.BlockSpec((64, HIDDEN), lambda i: (0, 0)),
            pl.BlockSpec((1, HIDDEN), lambda i: (0, 0)),
            pl.BlockSpec((HIDDEN, HIDDEN), lambda i: (0, 0)),
            pl.BlockSpec((1, HIDDEN), lambda i: (0, 0)),
        ],
        out_specs=pl.BlockSpec((2, _EB, HALF), lambda i: (0, i, 0)),
        out_shape=jax.ShapeDtypeStruct((2, N_EP, HALF), jnp.float32),
    )(eac, w1p, b1, w2, b2)


def _tc_emb(z_2d, emb_pad):
    def body(z_ref, emb_ref, out_ref):
        bi = lax.broadcasted_iota(jnp.int32, (_NB, 128), 1)
        oh = (bi == z_ref[...]).astype(jnp.float32)
        out_ref[...] = jnp.dot(oh, emb_ref[...],
                               preferred_element_type=jnp.float32)

    return pl.pallas_call(
        body,
        grid=(N_NODES // _NB,),
        in_specs=[
            pl.BlockSpec((_NB, 1), lambda i: (i, 0)),
            pl.BlockSpec((128, HIDDEN), lambda i: (0, 0)),
        ],
        out_specs=pl.BlockSpec((_NB, HIDDEN), lambda i: (i, 0)),
        out_shape=jax.ShapeDtypeStruct((N_NODES, HIDDEN), jnp.float32),
    )(z_2d, emb_pad)


def _tc_xf(h, w):
    def body(h_ref, w_ref, out_ref):
        xf = jnp.dot(h_ref[...], w_ref[...],
                     preferred_element_type=jnp.float32)
        out_ref[0] = xf[:, :HALF]
        out_ref[1] = xf[:, HALF:]

    return pl.pallas_call(
        body,
        grid=(N_NODES // _NB,),
        in_specs=[
            pl.BlockSpec((_NB, HIDDEN), lambda i: (i, 0)),
            pl.BlockSpec((HIDDEN, HIDDEN), lambda i: (0, 0)),
        ],
        out_specs=pl.BlockSpec((2, _NB, HALF), lambda i: (0, i, 0)),
        out_shape=jax.ShapeDtypeStruct((2, N_NODES, HALF), jnp.float32),
    )(h, w)


def _tc_out_mlp(agg3, h, w2, b2, lw, lb):
    def body(agg_ref, h_ref, w2_ref, b2_ref, lw_ref, lb_ref, out_ref):
        t = jnp.dot(agg_ref[0], w2_ref[:HALF, :],
                    preferred_element_type=jnp.float32)
        t = t + jnp.dot(agg_ref[1], w2_ref[HALF:, :],
                        preferred_element_type=jnp.float32)
        t = _ssp(t + b2_ref[...])
        t = jnp.dot(t, lw_ref[...], preferred_element_type=jnp.float32)
        out_ref[...] = h_ref[...] + t + lb_ref[...]

    return pl.pallas_call(
        body,
        grid=(N_NODES // _NB,),
        in_specs=[
            pl.BlockSpec((2, _NB, HALF), lambda i: (0, i, 0)),
            pl.BlockSpec((_NB, HIDDEN), lambda i: (i, 0)),
            pl.BlockSpec((HIDDEN, HIDDEN), lambda i: (0, 0)),
            pl.BlockSpec((1, HIDDEN), lambda i: (0, 0)),
            pl.BlockSpec((HIDDEN, HIDDEN), lambda i: (0, 0)),
            pl.BlockSpec((1, HIDDEN), lambda i: (0, 0)),
        ],
        out_specs=pl.BlockSpec((_NB, HIDDEN), lambda i: (i, 0)),
        out_shape=jax.ShapeDtypeStruct((N_NODES, HIDDEN), jnp.float32),
    )(agg3, h, w2, b2, lw, lb)


def _tc_pool(h, batch_row, pool_w, pool_b):
    nsteps = N_NODES // _NB

    def body(batch_ref, h_ref, pw_ref, pb_ref, out_ref, sums, cnt):
        i = pl.program_id(0)

        @pl.when(i == 0)
        def _init():
            sums[...] = jnp.zeros((NUM_GRAPHS, HIDDEN), jnp.float32)
            cnt[...] = jnp.zeros((NUM_GRAPHS, 128), jnp.float32)

        gi = lax.broadcasted_iota(jnp.int32, (NUM_GRAPHS, _NB), 0)
        oh = (gi == batch_ref[0]).astype(jnp.float32)
        sums[...] += jnp.dot(oh, h_ref[...],
                             preferred_element_type=jnp.float32)
        cnt[...] += jnp.dot(oh, jnp.ones((_NB, 128), jnp.float32),
                            preferred_element_type=jnp.float32)

        @pl.when(i == nsteps - 1)
        def _fin():
            pooled = sums[...] / jnp.maximum(cnt[:, 0:1], 1.0)
            out_ref[...] = jnp.dot(pooled, pw_ref[...],
                                   preferred_element_type=jnp.float32) \
                + pb_ref[...]

    return pl.pallas_call(
        body,
        grid=(nsteps,),
        in_specs=[
            pl.BlockSpec((1, 1, _NB), lambda i: (i, 0, 0)),
            pl.BlockSpec((_NB, HIDDEN), lambda i: (i, 0)),
            pl.BlockSpec((HIDDEN, HIDDEN), lambda i: (0, 0)),
            pl.BlockSpec((1, HIDDEN), lambda i: (0, 0)),
        ],
        out_specs=pl.BlockSpec((NUM_GRAPHS, HIDDEN), lambda i: (0, 0)),
        out_shape=jax.ShapeDtypeStruct((NUM_GRAPHS, HIDDEN), jnp.float32),
        scratch_shapes=[
            pltpu.VMEM((NUM_GRAPHS, HIDDEN), jnp.float32),
            pltpu.VMEM((NUM_GRAPHS, 128), jnp.float32),
        ],
    )(batch_row, h, pool_w, pool_b)


def kernel(z, pos, batch, edge_index, emb_table, mlp_w1, mlp_b1, mlp_w2,
           mlp_b2, conv1_w, conv2_w, conv2_b, lin_w, lin_b, pool_w, pool_b):
    src = edge_index[0].astype(jnp.int32)
    dst = edge_index[1].astype(jnp.int32)
    npad = N_EP - N_EDGES
    src_p = jnp.concatenate([src, jnp.zeros((npad,), jnp.int32)])
    dst_p = jnp.concatenate([dst, jnp.full((npad,), N_NODES, jnp.int32)])
    src2 = jnp.concatenate([src_p, src_p + N_NODES])
    pos_p = jnp.pad(pos.astype(jnp.float32), ((0, 0), (0, 1))).reshape(-1)

    d2 = _sc_geom(pos_p, src_p, dst_p)
    eac = _tc_edge_feats(d2.reshape(N_EP, 1))

    emb_pad = jnp.pad(emb_table, ((0, 128 - emb_table.shape[0]), (0, 0)))
    h = _tc_emb(z.reshape(N_NODES, 1).astype(jnp.int32), emb_pad)

    w1p = jnp.pad(mlp_w1, ((0, 0), (0, 64 - NUM_GAUSS), (0, 0)))

    for i in range(NUM_INTER):
        w3 = _tc_edge_mlp(eac, w1p[i], mlp_b1[i].reshape(1, HIDDEN),
                          mlp_w2[i], mlp_b2[i].reshape(1, HIDDEN))
        xf3 = _tc_xf(h, conv1_w[i])
        agg = _sc_cfconv(xf3.reshape(2 * N_NODES, HALF),
                         w3.reshape(2 * N_EP, HALF),
                         src2.reshape(-1, CHUNK), dst_p.reshape(-1, CHUNK))
        h = _tc_out_mlp(agg.reshape(2, N_NODES, HALF), h, conv2_w[i],
                        conv2_b[i].reshape(1, HIDDEN), lin_w[i],
                        lin_b[i].reshape(1, HIDDEN))

    return _tc_pool(h, batch.reshape(N_NODES // _NB, 1, _NB).astype(jnp.int32),
                    pool_w, pool_b.reshape(1, HIDDEN))


# speedup vs baseline: 2.5655x; 1.0820x over previous
import functools
import math

import jax
import jax.numpy as jnp
from jax import lax
from jax.experimental import pallas as pl
from jax.experimental.pallas import tpu as pltpu
from jax.experimental.pallas import tpu_sc as plsc

HIDDEN = 256
NUM_INTER = 6
NUM_GAUSS = 50
CUTOFF = 10.0
N_NODES = 10000
N_EDGES = 160000
NUM_GRAPHS = 64

NC = 2
NS = 16
LANES = 16
HALF = HIDDEN // 2
N_EP = 163840
CHUNK = 32
AGG_ROWS = 10240
_LN2 = math.log(2.0)


def _ssp(x):
    return jnp.maximum(x, 0.0) + jnp.log(1.0 + jnp.exp(-jnp.abs(x))) - _LN2


def _sc_geom(pos_p, src_p, dst_p):
    ept = N_EP // (NC * NS)
    mesh = plsc.VectorSubcoreMesh(core_axis_name="c", subcore_axis_name="s")

    @functools.partial(
        pl.kernel,
        out_type=jax.ShapeDtypeStruct((N_EP,), jnp.float32),
        mesh=mesh,
        compiler_params=pltpu.CompilerParams(needs_layout_passes=False),
    scratch_types=[
            pltpu.VMEM((4 * N_NODES,), jnp.float32),
            pltpu.VMEM((ept,), jnp.int32),
            pltpu.VMEM((ept,), jnp.int32),
            pltpu.VMEM((ept,), jnp.float32),
        ],
    )
    def k(pos_hbm, src_hbm, dst_hbm, out_hbm, pos_v, src_v, dst_v, out_v):
        cid = lax.axis_index("c")
        sid = lax.axis_index("s")
        wid = sid * NC + cid
        base = wid * ept
        pltpu.sync_copy(pos_hbm, pos_v)
        pltpu.sync_copy(src_hbm.at[pl.ds(base, ept)], src_v)
        pltpu.sync_copy(dst_hbm.at[pl.ds(base, ept)], dst_v)

        def body(g, carry):
            sl = pl.ds(g * LANES, LANES)
            sv = src_v[sl] * 4
            dv = dst_v[sl] * 4
            acc = jnp.full((LANES,), 1e-12, jnp.float32)
            for c in range(3):
                a = plsc.load_gather(pos_v, [sv + c])
                b = plsc.load_gather(pos_v, [dv + c])
                d = a - b
                acc = acc + d * d
            out_v[sl] = acc
            return carry

        lax.fori_loop(0, ept // LANES, body, 0)
        pltpu.sync_copy(out_v, out_hbm.at[pl.ds(base, ept)])

    return k(pos_p, src_p, dst_p)


def _sc_cfconv(xf_cat, w_cat, src2_2d, dst_2d):
    ept = N_EP // NS
    nch = ept // CHUNK
    zstripe = AGG_ROWS // NS
    ostripe = (N_NODES // NS) // 8 * 8
    mesh = plsc.VectorSubcoreMesh(core_axis_name="c", subcore_axis_name="s")

    @functools.partial(
        pl.kernel,
        out_type=jax.ShapeDtypeStruct((2 * N_NODES, HALF), jnp.float32),
        mesh=mesh,
        compiler_params=pltpu.CompilerParams(needs_layout_passes=False),
        scratch_types=[
            pltpu.VMEM((N_EP // NS // 128, 128), jnp.int32),
            pltpu.VMEM((3, CHUNK), jnp.int32),
            pltpu.VMEM((3, CHUNK, HALF), jnp.float32),
            pltpu.VMEM((3, CHUNK, HALF), jnp.float32),
            pltpu.VMEM_SHARED((AGG_ROWS, HALF), jnp.float32),
            pltpu.SemaphoreType.DMA,
            pltpu.SemaphoreType.DMA,
            pltpu.SemaphoreType.DMA,
            pltpu.SemaphoreType.DMA,
            pltpu.SemaphoreType.DMA,
            pltpu.SemaphoreType.DMA,
            pltpu.SemaphoreType.DMA,
            pltpu.SemaphoreType.DMA,
            pltpu.SemaphoreType.DMA,
            pltpu.SemaphoreType.DMA,
            pltpu.SemaphoreType.DMA,
            pltpu.SemaphoreType.DMA,
        ],
    )
    def k(xf_hbm, w_hbm, src_hbm, dst_hbm, out_hbm,
          src_all, dstb, rows_r, w_r, agg_sh,
          sg0, sg1, sg2, sw0, sw1, sw2, ss0, ss1, ss2, sd0, sd1, sd2):
        cid = lax.axis_index("c")
        sid = lax.axis_index("s")
        cpr = 128 // CHUNK
        irow = sid * nch
        grow = cid * (N_EP // 128) + sid * (nch // cpr)
        gbase = cid * N_EP + sid * ept
        sgs = (sg0, sg1, sg2)
        sws = (sw0, sw1, sw2)
        sss = (ss0, ss1, ss2)
        sds = (sd0, sd1, sd2)

        pltpu.sync_copy(src_hbm.at[pl.ds(grow, nch // cpr)], src_all)

        zvec = jnp.zeros((LANES,), jnp.float32)
        zbuf = rows_r.at[0]

        def zbody(j, carry):
            for f in range(HALF // LANES):
                zbuf[j, pl.ds(f * LANES, LANES)] = zvec
            return carry

        lax.fori_loop(0, CHUNK, zbody, 0)
        for q in range(zstripe // CHUNK):
            pltpu.sync_copy(
                zbuf, agg_sh.at[pl.ds(sid * zstripe + q * CHUNK, CHUNK)])
        plsc.subcore_barrier()

        def _src_idx(kk):
            return src_all.at[kk // cpr, pl.ds((kk % cpr) * CHUNK, CHUNK)]

        def g_issue(kk, s):
            pltpu.async_copy(xf_hbm.at[_src_idx(kk)], rows_r.at[s], sgs[s])
            pltpu.async_copy(w_hbm.at[pl.ds(gbase + kk * CHUNK, CHUNK)],
                             w_r.at[s], sws[s])
            pltpu.async_copy(dst_hbm.at[irow + kk], dstb.at[s], sds[s])

        def g_wait(kk, s):
            pltpu.make_async_copy(
                xf_hbm.at[_src_idx(kk)], rows_r.at[s], sgs[s]).wait()
            pltpu.make_async_copy(
                w_hbm.at[pl.ds(gbase + kk * CHUNK, CHUNK)],
                w_r.at[s], sws[s]).wait()

        def s_issue(kk, s):
            pltpu.make_async_copy(
                dst_hbm.at[irow + kk], dstb.at[s], sds[s]).wait()
            pltpu.async_copy(rows_r.at[s], agg_sh.at[dstb.at[s]],
                             sss[s], add=True)

        def s_wait(kk, s):
            pltpu.make_async_copy(
                rows_r.at[s], agg_sh.at[dstb.at[s]], sss[s]).wait()

        def mul(s):
            rv = rows_r.at[s]
            wv = w_r.at[s]

            def mbody(j, carry2):
                for u in range(2):
                    for f in range(HALF // LANES):
                        sl = pl.ds(f * LANES, LANES)
                        rv[2 * j + u, sl] = rv[2 * j + u, sl] * wv[2 * j + u, sl]
                return carry2

            lax.fori_loop(0, CHUNK // 2, mbody, 0)

        def sub(c, k_, guard_first):
            s = k_ % 3
            g_wait(c, s)
            mul(s)
            s_issue(c, s)
            s2 = (k_ + 2) % 3
            if guard_first:
                @pl.when(c > 0)
                def _w():
                    s_wait(c - 1, s2)
            else:
                s_wait(c - 1, s2)
            g_issue(c + 2, s2)

        g_issue(0, 0)
        g_issue(1, 1)

        def tri_body(i, carry):
            c0 = 3 * i
            sub(c0, 0, True)
            sub(c0 + 1, 1, False)
            sub(c0 + 2, 2, False)
            return carry

        lax.fori_loop(0, (nch - 2) // 3, tri_body, 0)

        for (c, s) in ((nch - 2, (nch - 2) % 3), (nch - 1, (nch - 1) % 3)):
            g_wait(c, s)
            mul(s)
            s_issue(c, s)

        s_wait(nch - 3, (nch - 3) % 3)
        s_wait(nch - 2, (nch - 2) % 3)
        s_wait(nch - 1, (nch - 1) % 3)
        plsc.subcore_barrier()

        pltpu.sync_copy(
            agg_sh.at[pl.ds(sid * ostripe, ostripe)],
            out_hbm.at[pl.ds(cid * N_NODES + sid * ostripe, ostripe)])

        @pl.when(sid == 0)
        def _tail():
            pltpu.sync_copy(
                agg_sh.at[pl.ds(NS * ostripe, N_NODES - NS * ostripe)],
                out_hbm.at[pl.ds(cid * N_NODES + NS * ostripe,
                                 N_NODES - NS * ostripe)])

    return k(xf_cat, w_cat, src2_2d, dst_2d)


_EB = 2048
_NB = 2000


def _tc_edge_feats(d2_2d):
    step = CUTOFF / (NUM_GAUSS - 1)
    coeff = -0.5 / step ** 2

    def body(d2_ref, out_ref):
        d2 = d2_ref[...]
        dist = jnp.sqrt(d2)
        kii = lax.broadcasted_iota(jnp.int32, (_EB, 64), 1)
        kidx = kii.astype(jnp.float32)
        gauss = jnp.exp(coeff * (dist - kidx * step) ** 2)
        cc = 0.5 * (jnp.cos(dist * (math.pi / CUTOFF)) + 1.0)
        out_ref[...] = jnp.where(kii == NUM_GAUSS, cc, gauss)

    return pl.pallas_call(
        body,
        grid=(N_EP // _EB,),
        in_specs=[pl.BlockSpec((_EB, 1), lambda i: (i, 0))],
        out_specs=pl.BlockSpec((_EB, 64), lambda i: (i, 0)),
        out_shape=jax.ShapeDtypeStruct((N_EP, 64), jnp.float32),
    )(d2_2d)


def _tc_edge_mlp(eac, w1p, b1, w2, b2):
    def body(ea_ref, w1_ref, b1_ref, w2_ref, b2_ref, out_ref):
        ea = ea_ref[...]
        h1 = jnp.dot(ea, w1_ref[...], preferred_element_type=jnp.float32)
        h1 = _ssp(h1 + b1_ref[...])
        w = jnp.dot(h1, w2_ref[...], preferred_element_type=jnp.float32)
        w = (w + b2_ref[...]) * ea[:, NUM_GAUSS:NUM_GAUSS + 1]
        out_ref[0] = w[:, :HALF]
        out_ref[1] = w[:, HALF:]

    return pl.pallas_call(
        body,
        grid=(N_EP // _EB,),
        in_specs=[
            pl.BlockSpec((_EB, 64), lambda i: (i, 0)),
            pl.BlockSpec((64, HIDDEN), lambda i: (0, 0)),
            pl.BlockSpec((1, HIDDEN), lambda i: (0, 0)),
            pl.BlockSpec((HIDDEN, HIDDEN), lambda i: (0, 0)),
            pl.BlockSpec((1, HIDDEN), lambda i: (0, 0)),
        ],
        out_specs=pl.BlockSpec((2, _EB, HALF), lambda i: (0, i, 0)),
        out_shape=jax.ShapeDtypeStruct((2, N_EP, HALF), jnp.float32),
    )(eac, w1p, b1, w2, b2)


def _tc_emb(z_2d, emb_pad):
    def body(z_ref, emb_ref, out_ref):
        bi = lax.broadcasted_iota(jnp.int32, (_NB, 128), 1)
        oh = (bi == z_ref[...]).astype(jnp.float32)
        out_ref[...] = jnp.dot(oh, emb_ref[...],
                               preferred_element_type=jnp.float32)

    return pl.pallas_call(
        body,
        grid=(N_NODES // _NB,),
        in_specs=[
            pl.BlockSpec((_NB, 1), lambda i: (i, 0)),
            pl.BlockSpec((128, HIDDEN), lambda i: (0, 0)),
        ],
        out_specs=pl.BlockSpec((_NB, HIDDEN), lambda i: (i, 0)),
        out_shape=jax.ShapeDtypeStruct((N_NODES, HIDDEN), jnp.float32),
    )(z_2d, emb_pad)


def _tc_xf(h, w):
    def body(h_ref, w_ref, out_ref):
        xf = jnp.dot(h_ref[...], w_ref[...],
                     preferred_element_type=jnp.float32)
        out_ref[0] = xf[:, :HALF]
        out_ref[1] = xf[:, HALF:]

    return pl.pallas_call(
        body,
        grid=(N_NODES // _NB,),
        in_specs=[
            pl.BlockSpec((_NB, HIDDEN), lambda i: (i, 0)),
            pl.BlockSpec((HIDDEN, HIDDEN), lambda i: (0, 0)),
        ],
        out_specs=pl.BlockSpec((2, _NB, HALF), lambda i: (0, i, 0)),
        out_shape=jax.ShapeDtypeStruct((2, N_NODES, HALF), jnp.float32),
    )(h, w)


def _tc_out_mlp(agg3, h, w2, b2, lw, lb):
    def body(agg_ref, h_ref, w2_ref, b2_ref, lw_ref, lb_ref, out_ref):
        t = jnp.dot(agg_ref[0], w2_ref[:HALF, :],
                    preferred_element_type=jnp.float32)
        t = t + jnp.dot(agg_ref[1], w2_ref[HALF:, :],
                        preferred_element_type=jnp.float32)
        t = _ssp(t + b2_ref[...])
        t = jnp.dot(t, lw_ref[...], preferred_element_type=jnp.float32)
        out_ref[...] = h_ref[...] + t + lb_ref[...]

    return pl.pallas_call(
        body,
        grid=(N_NODES // _NB,),
        in_specs=[
            pl.BlockSpec((2, _NB, HALF), lambda i: (0, i, 0)),
            pl.BlockSpec((_NB, HIDDEN), lambda i: (i, 0)),
            pl.BlockSpec((HIDDEN, HIDDEN), lambda i: (0, 0)),
            pl.BlockSpec((1, HIDDEN), lambda i: (0, 0)),
            pl.BlockSpec((HIDDEN, HIDDEN), lambda i: (0, 0)),
            pl.BlockSpec((1, HIDDEN), lambda i: (0, 0)),
        ],
        out_specs=pl.BlockSpec((_NB, HIDDEN), lambda i: (i, 0)),
        out_shape=jax.ShapeDtypeStruct((N_NODES, HIDDEN), jnp.float32),
    )(agg3, h, w2, b2, lw, lb)


def _tc_pool(h, batch_row, pool_w, pool_b):
    nsteps = N_NODES // _NB

    def body(batch_ref, h_ref, pw_ref, pb_ref, out_ref, sums, cnt):
        i = pl.program_id(0)

        @pl.when(i == 0)
        def _init():
            sums[...] = jnp.zeros((NUM_GRAPHS, HIDDEN), jnp.float32)
            cnt[...] = jnp.zeros((NUM_GRAPHS, 128), jnp.float32)

        gi = lax.broadcasted_iota(jnp.int32, (NUM_GRAPHS, _NB), 0)
        oh = (gi == batch_ref[0]).astype(jnp.float32)
        sums[...] += jnp.dot(oh, h_ref[...],
                             preferred_element_type=jnp.float32)
        cnt[...] += jnp.dot(oh, jnp.ones((_NB, 128), jnp.float32),
                            preferred_element_type=jnp.float32)

        @pl.when(i == nsteps - 1)
        def _fin():
            pooled = sums[...] / jnp.maximum(cnt[:, 0:1], 1.0)
            out_ref[...] = jnp.dot(pooled, pw_ref[...],
                                   preferred_element_type=jnp.float32) \
                + pb_ref[...]

    return pl.pallas_call(
        body,
        grid=(nsteps,),
        in_specs=[
            pl.BlockSpec((1, 1, _NB), lambda i: (i, 0, 0)),
            pl.BlockSpec((_NB, HIDDEN), lambda i: (i, 0)),
            pl.BlockSpec((HIDDEN, HIDDEN), lambda i: (0, 0)),
            pl.BlockSpec((1, HIDDEN), lambda i: (0, 0)),
        ],
        out_specs=pl.BlockSpec((NUM_GRAPHS, HIDDEN), lambda i: (0, 0)),
        out_shape=jax.ShapeDtypeStruct((NUM_GRAPHS, HIDDEN), jnp.float32),
        scratch_shapes=[
            pltpu.VMEM((NUM_GRAPHS, HIDDEN), jnp.float32),
            pltpu.VMEM((NUM_GRAPHS, 128), jnp.float32),
        ],
    )(batch_row, h, pool_w, pool_b)


def kernel(z, pos, batch, edge_index, emb_table, mlp_w1, mlp_b1, mlp_w2,
           mlp_b2, conv1_w, conv2_w, conv2_b, lin_w, lin_b, pool_w, pool_b):
    src = edge_index[0].astype(jnp.int32)
    dst = edge_index[1].astype(jnp.int32)
    npad = N_EP - N_EDGES
    src_p = jnp.concatenate([src, jnp.zeros((npad,), jnp.int32)])
    dst_p = jnp.concatenate([dst, jnp.full((npad,), N_NODES, jnp.int32)])
    src2 = jnp.concatenate([src_p, src_p + N_NODES])
    pos_p = jnp.pad(pos.astype(jnp.float32), ((0, 0), (0, 1))).reshape(-1)

    d2 = _sc_geom(pos_p, src_p, dst_p)
    eac = _tc_edge_feats(d2.reshape(N_EP, 1))

    emb_pad = jnp.pad(emb_table, ((0, 128 - emb_table.shape[0]), (0, 0)))
    h = _tc_emb(z.reshape(N_NODES, 1).astype(jnp.int32), emb_pad)

    w1p = jnp.pad(mlp_w1, ((0, 0), (0, 64 - NUM_GAUSS), (0, 0)))

    for i in range(NUM_INTER):
        w3 = _tc_edge_mlp(eac, w1p[i], mlp_b1[i].reshape(1, HIDDEN),
                          mlp_w2[i], mlp_b2[i].reshape(1, HIDDEN))
        xf3 = _tc_xf(h, conv1_w[i])
        agg = _sc_cfconv(xf3.reshape(2 * N_NODES, HALF),
                         w3.reshape(2 * N_EP, HALF),
                         src2.reshape(-1, 128), dst_p.reshape(-1, CHUNK))
        h = _tc_out_mlp(agg.reshape(2, N_NODES, HALF), h, conv2_w[i],
                        conv2_b[i].reshape(1, HIDDEN), lin_w[i],
                        lin_b[i].reshape(1, HIDDEN))

    return _tc_pool(h, batch.reshape(N_NODES // _NB, 1, _NB).astype(jnp.int32),
                    pool_w, pool_b.reshape(1, HIDDEN))


# speedup vs baseline: 2.5678x; 1.0009x over previous
import functools
import math

import jax
import jax.numpy as jnp
from jax import lax
from jax.experimental import pallas as pl
from jax.experimental.pallas import tpu as pltpu
from jax.experimental.pallas import tpu_sc as plsc

HIDDEN = 256
NUM_INTER = 6
NUM_GAUSS = 50
CUTOFF = 10.0
N_NODES = 10000
N_EDGES = 160000
NUM_GRAPHS = 64

NC = 2
NS = 16
LANES = 16
HALF = HIDDEN // 2
N_EP = 163840
CHUNK = 32
AGG_ROWS = 10240
_LN2 = math.log(2.0)


def _ssp(x):
    return jnp.maximum(x, 0.0) + jnp.log(1.0 + jnp.exp(-jnp.abs(x))) - _LN2


def _sc_geom(pos_p, src_p, dst_p):
    ept = N_EP // (NC * NS)
    mesh = plsc.VectorSubcoreMesh(core_axis_name="c", subcore_axis_name="s")

    @functools.partial(
        pl.kernel,
        out_type=jax.ShapeDtypeStruct((N_EP,), jnp.float32),
        mesh=mesh,
        compiler_params=pltpu.CompilerParams(needs_layout_passes=False),
    scratch_types=[
            pltpu.VMEM((4 * N_NODES,), jnp.float32),
            pltpu.VMEM((ept,), jnp.int32),
            pltpu.VMEM((ept,), jnp.int32),
            pltpu.VMEM((ept,), jnp.float32),
        ],
    )
    def k(pos_hbm, src_hbm, dst_hbm, out_hbm, pos_v, src_v, dst_v, out_v):
        cid = lax.axis_index("c")
        sid = lax.axis_index("s")
        wid = sid * NC + cid
        base = wid * ept
        pltpu.sync_copy(pos_hbm, pos_v)
        pltpu.sync_copy(src_hbm.at[pl.ds(base, ept)], src_v)
        pltpu.sync_copy(dst_hbm.at[pl.ds(base, ept)], dst_v)

        def body(g, carry):
            sl = pl.ds(g * LANES, LANES)
            sv = src_v[sl] * 4
            dv = dst_v[sl] * 4
            acc = jnp.full((LANES,), 1e-12, jnp.float32)
            for c in range(3):
                a = plsc.load_gather(pos_v, [sv + c])
                b = plsc.load_gather(pos_v, [dv + c])
                d = a - b
                acc = acc + d * d
            out_v[sl] = acc
            return carry

        lax.fori_loop(0, ept // LANES, body, 0)
        pltpu.sync_copy(out_v, out_hbm.at[pl.ds(base, ept)])

    return k(pos_p, src_p, dst_p)


def _sc_cfconv(xf_cat, w_cat, src2_2d, dst_2d):
    ept = N_EP // NS
    nch = ept // CHUNK
    zstripe = AGG_ROWS // NS
    ostripe = (N_NODES // NS) // 8 * 8
    mesh = plsc.VectorSubcoreMesh(core_axis_name="c", subcore_axis_name="s")

    @functools.partial(
        pl.kernel,
        out_type=jax.ShapeDtypeStruct((2 * N_NODES, HALF), jnp.float32),
        mesh=mesh,
        compiler_params=pltpu.CompilerParams(needs_layout_passes=False),
        scratch_types=[
            pltpu.VMEM((N_EP // NS // 128, 128), jnp.int32),
            pltpu.VMEM((3, CHUNK), jnp.int32),
            pltpu.VMEM((3, CHUNK, HALF), jnp.float32),
            pltpu.VMEM((3, CHUNK, HALF), jnp.float32),
            pltpu.VMEM_SHARED((AGG_ROWS, HALF), jnp.float32),
            pltpu.SemaphoreType.DMA,
            pltpu.SemaphoreType.DMA,
            pltpu.SemaphoreType.DMA,
            pltpu.SemaphoreType.DMA,
            pltpu.SemaphoreType.DMA,
            pltpu.SemaphoreType.DMA,
            pltpu.SemaphoreType.DMA,
            pltpu.SemaphoreType.DMA,
            pltpu.SemaphoreType.DMA,
            pltpu.SemaphoreType.DMA,
            pltpu.SemaphoreType.DMA,
            pltpu.SemaphoreType.DMA,
        ],
    )
    def k(xf_hbm, w_hbm, src_hbm, dst_hbm, out_hbm,
          src_all, dstb, rows_r, w_r, agg_sh,
          sg0, sg1, sg2, sw0, sw1, sw2, ss0, ss1, ss2, sd0, sd1, sd2):
        cid = lax.axis_index("c")
        sid = lax.axis_index("s")
        cpr = 128 // CHUNK
        irow = sid * nch
        grow = cid * (N_EP // 128) + sid * (nch // cpr)
        gbase = cid * N_EP + sid * ept
        sgs = (sg0, sg1, sg2)
        sws = (sw0, sw1, sw2)
        sss = (ss0, ss1, ss2)
        sds = (sd0, sd1, sd2)

        pltpu.sync_copy(src_hbm.at[pl.ds(grow, nch // cpr)], src_all)

        zvec = jnp.zeros((LANES,), jnp.float32)
        zbuf = rows_r.at[0]

        def zbody(j, carry):
            for f in range(HALF // LANES):
                zbuf[j, pl.ds(f * LANES, LANES)] = zvec
            return carry

        lax.fori_loop(0, CHUNK, zbody, 0)
        for q in range(zstripe // CHUNK):
            pltpu.sync_copy(
                zbuf, agg_sh.at[pl.ds(sid * zstripe + q * CHUNK, CHUNK)])
        plsc.subcore_barrier()

        def _src_idx(kk):
            return src_all.at[kk // cpr, pl.ds((kk % cpr) * CHUNK, CHUNK)]

        def g_issue(kk, s):
            pltpu.async_copy(xf_hbm.at[_src_idx(kk)], rows_r.at[s], sgs[s])
            pltpu.async_copy(w_hbm.at[pl.ds(gbase + kk * CHUNK, CHUNK)],
                             w_r.at[s], sws[s])
            pltpu.async_copy(dst_hbm.at[irow + kk], dstb.at[s], sds[s])

        def g_wait(kk, s):
            pltpu.make_async_copy(
                xf_hbm.at[_src_idx(kk)], rows_r.at[s], sgs[s]).wait()
            pltpu.make_async_copy(
                w_hbm.at[pl.ds(gbase + kk * CHUNK, CHUNK)],
                w_r.at[s], sws[s]).wait()

        def s_issue(kk, s):
            pltpu.make_async_copy(
                dst_hbm.at[irow + kk], dstb.at[s], sds[s]).wait()
            pltpu.async_copy(rows_r.at[s], agg_sh.at[dstb.at[s]],
                             sss[s], add=True)

        def s_wait(kk, s):
            pltpu.make_async_copy(
                rows_r.at[s], agg_sh.at[dstb.at[s]], sss[s]).wait()

        def mul(s):
            rv = rows_r.at[s]
            wv = w_r.at[s]

            def mbody(j, carry2):
                for u in range(2):
                    for f in range(HALF // LANES):
                        sl = pl.ds(f * LANES, LANES)
                        rv[2 * j + u, sl] = rv[2 * j + u, sl] * wv[2 * j + u, sl]
                return carry2

            lax.fori_loop(0, CHUNK // 2, mbody, 0)

        def sub(c, k_, guard_first):
            s = k_ % 3
            g_wait(c, s)
            mul(s)
            s_issue(c, s)
            s2 = (k_ + 2) % 3
            if guard_first:
                @pl.when(c > 0)
                def _w():
                    s_wait(c - 1, s2)
            else:
                s_wait(c - 1, s2)
            g_issue(c + 2, s2)

        g_issue(0, 0)
        g_issue(1, 1)

        def tri_body(i, carry):
            c0 = 3 * i
            sub(c0, 0, True)
            sub(c0 + 1, 1, False)
            sub(c0 + 2, 2, False)
            return carry

        lax.fori_loop(0, (nch - 2) // 3, tri_body, 0)

        for (c, s) in ((nch - 2, (nch - 2) % 3), (nch - 1, (nch - 1) % 3)):
            g_wait(c, s)
            mul(s)
            s_issue(c, s)

        s_wait(nch - 3, (nch - 3) % 3)
        s_wait(nch - 2, (nch - 2) % 3)
        s_wait(nch - 1, (nch - 1) % 3)
        plsc.subcore_barrier()

        pltpu.sync_copy(
            agg_sh.at[pl.ds(sid * ostripe, ostripe)],
            out_hbm.at[pl.ds(cid * N_NODES + sid * ostripe, ostripe)])

        @pl.when(sid == 0)
        def _tail():
            pltpu.sync_copy(
                agg_sh.at[pl.ds(NS * ostripe, N_NODES - NS * ostripe)],
                out_hbm.at[pl.ds(cid * N_NODES + NS * ostripe,
                                 N_NODES - NS * ostripe)])

    return k(xf_cat, w_cat, src2_2d, dst_2d)


_EB = 2048
_NB = 2000


def _tc_edge_feats(d2_2d):
    step = CUTOFF / (NUM_GAUSS - 1)
    coeff = -0.5 / step ** 2

    def body(d2_ref, out_ref):
        d2 = d2_ref[...]
        dist = jnp.sqrt(d2)
        kii = lax.broadcasted_iota(jnp.int32, (_EB, 64), 1)
        kidx = kii.astype(jnp.float32)
        gauss = jnp.exp(coeff * (dist - kidx * step) ** 2)
        cc = 0.5 * (jnp.cos(dist * (math.pi / CUTOFF)) + 1.0)
        out_ref[...] = jnp.where(kii == NUM_GAUSS, cc, gauss)

    return pl.pallas_call(
        body,
        grid=(N_EP // _EB,),
        in_specs=[pl.BlockSpec((_EB, 1), lambda i: (i, 0))],
        out_specs=pl.BlockSpec((_EB, 64), lambda i: (i, 0)),
        out_shape=jax.ShapeDtypeStruct((N_EP, 64), jnp.float32),
    )(d2_2d)


def _tc_edge_mlp(eac, w1p, b1, w2, b2):
    def body(ea_ref, w1_ref, b1_ref, w2_ref, b2_ref, out_ref):
        ea = ea_ref[...]
        h1 = jnp.dot(ea, w1_ref[...], preferred_element_type=jnp.float32)
        h1 = _ssp(h1 + b1_ref[...])
        w = jnp.dot(h1, w2_ref[...], preferred_element_type=jnp.float32)
        w = (w + b2_ref[...]) * ea[:, NUM_GAUSS:NUM_GAUSS + 1]
        out_ref[0] = w[:, :HALF]
        out_ref[1] = w[:, HALF:]

    return pl.pallas_call(
        body,
        grid=(N_EP // _EB,),
        in_specs=[
            pl.BlockSpec((_EB, 64), lambda i: (i, 0)),
            pl.BlockSpec((64, HIDDEN), lambda i: (0, 0)),
            pl.BlockSpec((1, HIDDEN), lambda i: (0, 0)),
            pl.BlockSpec((HIDDEN, HIDDEN), lambda i: (0, 0)),
            pl.BlockSpec((1, HIDDEN), lambda i: (0, 0)),
        ],
        out_specs=pl.BlockSpec((2, _EB, HALF), lambda i: (0, i, 0)),
        out_shape=jax.ShapeDtypeStruct((2, N_EP, HALF), jnp.float32),
    )(eac, w1p, b1, w2, b2)


def _tc_emb(z_2d, emb_pad):
    def body(z_ref, emb_ref, out_ref):
        bi = lax.broadcasted_iota(jnp.int32, (_NB, 128), 1)
        oh = (bi == z_ref[...]).astype(jnp.float32)
        out_ref[...] = jnp.dot(oh, emb_ref[...],
                               preferred_element_type=jnp.float32)

    return pl.pallas_call(
        body,
        grid=(N_NODES // _NB,),
        in_specs=[
            pl.BlockSpec((_NB, 1), lambda i: (i, 0)),
            pl.BlockSpec((128, HIDDEN), lambda i: (0, 0)),
        ],
        out_specs=pl.BlockSpec((_NB, HIDDEN), lambda i: (i, 0)),
        out_shape=jax.ShapeDtypeStruct((N_NODES, HIDDEN), jnp.float32),
    )(z_2d, emb_pad)


def _tc_xf(h, w):
    def body(h_ref, w_ref, out_ref):
        xf = jnp.dot(h_ref[...], w_ref[...],
                     preferred_element_type=jnp.float32)
        out_ref[0] = xf[:, :HALF]
        out_ref[1] = xf[:, HALF:]

    return pl.pallas_call(
        body,
        grid=(N_NODES // _NB,),
        in_specs=[
            pl.BlockSpec((_NB, HIDDEN), lambda i: (i, 0)),
            pl.BlockSpec((HIDDEN, HIDDEN), lambda i: (0, 0)),
        ],
        out_specs=pl.BlockSpec((2, _NB, HALF), lambda i: (0, i, 0)),
        out_shape=jax.ShapeDtypeStruct((2, N_NODES, HALF), jnp.float32),
    )(h, w)


def _tc_out_mlp(agg3, h, w2, b2, lw, lb):
    def body(agg_ref, h_ref, w2_ref, b2_ref, lw_ref, lb_ref, out_ref):
        t = jnp.dot(agg_ref[0], w2_ref[:HALF, :],
                    preferred_element_type=jnp.float32)
        t = t + jnp.dot(agg_ref[1], w2_ref[HALF:, :],
                        preferred_element_type=jnp.float32)
        t = _ssp(t + b2_ref[...])
        t = jnp.dot(t, lw_ref[...], preferred_element_type=jnp.float32)
        out_ref[...] = h_ref[...] + t + lb_ref[...]

    return pl.pallas_call(
        body,
        grid=(N_NODES // _NB,),
        in_specs=[
            pl.BlockSpec((2, _NB, HALF), lambda i: (0, i, 0)),
            pl.BlockSpec((_NB, HIDDEN), lambda i: (i, 0)),
            pl.BlockSpec((HIDDEN, HIDDEN), lambda i: (0, 0)),
            pl.BlockSpec((1, HIDDEN), lambda i: (0, 0)),
            pl.BlockSpec((HIDDEN, HIDDEN), lambda i: (0, 0)),
            pl.BlockSpec((1, HIDDEN), lambda i: (0, 0)),
        ],
        out_specs=pl.BlockSpec((_NB, HIDDEN), lambda i: (i, 0)),
        out_shape=jax.ShapeDtypeStruct((N_NODES, HIDDEN), jnp.float32),
    )(agg3, h, w2, b2, lw, lb)


def _tc_pool(h, batch_row, pool_w, pool_b):
    nsteps = N_NODES // _NB

    def body(batch_ref, h_ref, pw_ref, pb_ref, out_ref, sums, cnt):
        i = pl.program_id(0)

        @pl.when(i == 0)
        def _init():
            sums[...] = jnp.zeros((NUM_GRAPHS, HIDDEN), jnp.float32)
            cnt[...] = jnp.zeros((NUM_GRAPHS, 128), jnp.float32)

        gi = lax.broadcasted_iota(jnp.int32, (NUM_GRAPHS, _NB), 0)
        oh = (gi == batch_ref[0]).astype(jnp.float32)
        sums[...] += jnp.dot(oh, h_ref[...],
                             preferred_element_type=jnp.float32)
        cnt[...] += jnp.dot(oh, jnp.ones((_NB, 128), jnp.float32),
                            preferred_element_type=jnp.float32)

        @pl.when(i == nsteps - 1)
        def _fin():
            pooled = sums[...] / jnp.maximum(cnt[:, 0:1], 1.0)
            out_ref[...] = jnp.dot(pooled, pw_ref[...],
                                   preferred_element_type=jnp.float32) \
                + pb_ref[...]

    return pl.pallas_call(
        body,
        grid=(nsteps,),
        in_specs=[
            pl.BlockSpec((1, 1, _NB), lambda i: (i, 0, 0)),
            pl.BlockSpec((_NB, HIDDEN), lambda i: (i, 0)),
            pl.BlockSpec((HIDDEN, HIDDEN), lambda i: (0, 0)),
            pl.BlockSpec((1, HIDDEN), lambda i: (0, 0)),
        ],
        out_specs=pl.BlockSpec((NUM_GRAPHS, HIDDEN), lambda i: (0, 0)),
        out_shape=jax.ShapeDtypeStruct((NUM_GRAPHS, HIDDEN), jnp.float32),
        scratch_shapes=[
            pltpu.VMEM((NUM_GRAPHS, HIDDEN), jnp.float32),
            pltpu.VMEM((NUM_GRAPHS, 128), jnp.float32),
        ],
    )(batch_row, h, pool_w, pool_b)


def kernel(z, pos, batch, edge_index, emb_table, mlp_w1, mlp_b1, mlp_w2,
           mlp_b2, conv1_w, conv2_w, conv2_b, lin_w, lin_b, pool_w, pool_b):
    src = edge_index[0].astype(jnp.int32)
    dst = edge_index[1].astype(jnp.int32)
    npad = N_EP - N_EDGES
    src_p = jnp.concatenate([src, jnp.zeros((npad,), jnp.int32)])
    dst_p = jnp.concatenate([dst, jnp.full((npad,), N_NODES, jnp.int32)])
    src2 = jnp.concatenate([src_p, src_p + N_NODES])
    pos_p = jnp.pad(pos.astype(jnp.float32), ((0, 0), (0, 1))).reshape(-1)

    d2 = _sc_geom(pos_p, src_p, dst_p)
    eac = _tc_edge_feats(d2.reshape(N_EP, 1))

    emb_pad = jnp.pad(emb_table, ((0, 128 - emb_table.shape[0]), (0, 0)))
    h = _tc_emb(z.reshape(N_NODES, 1).astype(jnp.int32), emb_pad)

    w1p = jnp.pad(mlp_w1, ((0, 0), (0, 64 - NUM_GAUSS), (0, 0)))

    w3s = [_tc_edge_mlp(eac, w1p[i], mlp_b1[i].reshape(1, HIDDEN),
                        mlp_w2[i], mlp_b2[i].reshape(1, HIDDEN))
           for i in range(NUM_INTER)]

    for i in range(NUM_INTER):
        w3 = w3s[i]
        xf3 = _tc_xf(h, conv1_w[i])
        agg = _sc_cfconv(xf3.reshape(2 * N_NODES, HALF),
                         w3.reshape(2 * N_EP, HALF),
                         src2.reshape(-1, 128), dst_p.reshape(-1, CHUNK))
        h = _tc_out_mlp(agg.reshape(2, N_NODES, HALF), h, conv2_w[i],
                        conv2_b[i].reshape(1, HIDDEN), lin_w[i],
                        lin_b[i].reshape(1, HIDDEN))

    return _tc_pool(h, batch.reshape(N_NODES // _NB, 1, _NB).astype(jnp.int32),
                    pool_w, pool_b.reshape(1, HIDDEN))
